# baseline (speedup 1.0000x reference)
import jax
import jax.numpy as jnp
from jax import lax
from jax.experimental import pallas as pl
from jax.experimental.pallas import tpu as pltpu

B, N, D, H, A = 1024, 16, 128, 128, 4
BS = 128
G = B // BS
EPS = 1e-5


def _ln(x, g, b):
    mu = jnp.mean(x, axis=-1, keepdims=True)
    var = jnp.mean((x - mu) ** 2, axis=-1, keepdims=True)
    return (x - mu) * lax.rsqrt(var + EPS) * g + b


def _edge_tail(E, J, ge, bbe):
    mu = jnp.dot(E, J, preferred_element_type=jnp.float32)
    d = E - mu
    var = jnp.mean(d * d, axis=-1, keepdims=True)
    return jax.nn.relu(d * (lax.rsqrt(var + EPS) * ge) + bbe)


def _block_kernel(x_ref, act_ref, we1a_ref, we1b_ref, be1_ref, we2_ref,
                  be2_ref, ge_ref, bbe_ref, we3_ref, be3_ref, wn1x_ref,
                  wn1a_ref, wn1g_ref, bn1_ref, wn2_ref, bn2_ref, gn_ref,
                  bbn_ref, wn3_ref, bn3_ref, j_ref, out_ref):
    f32 = jnp.float32
    Xf = x_ref[...].reshape(BS * N, D)
    J = j_ref[...]
    ge, bbe = ge_ref[...], bbe_ref[...]
    P = jnp.dot(Xf, we1a_ref[...], preferred_element_type=f32)
    Q = jnp.dot(Xf, we1b_ref[...], preferred_element_type=f32) + be1_ref[...]
    F = jax.nn.relu(P.reshape(BS, 1, N, H) + Q.reshape(BS, N, 1, H))
    E = jnp.dot(F.reshape(BS * N * N, H), we2_ref[...],
                preferred_element_type=f32) + be2_ref[...]
    T = _edge_tail(E, J, ge, bbe)
    Ssum = jnp.sum(T.reshape(BS, N, N, H), axis=1).reshape(BS * N, H)
    Ed = jnp.dot(jax.nn.relu(P + Q), we2_ref[...],
                 preferred_element_type=f32) + be2_ref[...]
    Td = _edge_tail(Ed, J, ge, bbe)
    S = Ssum - Td
    agg = jnp.dot(S, we3_ref[...], preferred_element_type=f32) \
        + (N - 1) * be3_ref[...]
    a = act_ref[...].reshape(BS, 1, 1)
    kk = lax.broadcasted_iota(jnp.int32, (BS, N, 8), 2)
    code = lax.broadcasted_iota(jnp.int32, (BS, N, 8), 1) * A + kk
    oh = jnp.where((code == a) & (kk < A), 1.0, 0.0).reshape(BS * N, 8)
    h = (jnp.dot(Xf, wn1x_ref[...], preferred_element_type=f32)
         + jnp.dot(oh, wn1a_ref[...], preferred_element_type=f32)
         + jnp.dot(agg, wn1g_ref[...], preferred_element_type=f32))
    h = jax.nn.relu(h + bn1_ref[...])
    h = jnp.dot(h, wn2_ref[...], preferred_element_type=f32) + bn2_ref[...]
    h = jax.nn.relu(_ln(h, gn_ref[...], bbn_ref[...]))
    out = jnp.dot(h, wn3_ref[...], preferred_element_type=f32) + bn3_ref[...]
    out_ref[...] = out.reshape(BS, N, D)


def _full(shape):
    nd = len(shape)
    return pl.BlockSpec(shape, lambda b: (0,) * nd)


def kernel(states, action, We1, be1, We2, be2, ge, bbe, We3, be3,
           Wn1, bn1, Wn2, bn2, gn, bbn, Wn3, bn3):
    act = action.reshape(G, 1, BS)
    We1a, We1b = We1[:D], We1[D:]
    Wn1x = Wn1[:D]
    Wn1a = jnp.zeros((8, H), jnp.float32).at[:A].set(Wn1[D:D + A])
    Wn1g = Wn1[D + A:]
    Jmat = jnp.full((H, H), 1.0 / H, jnp.float32)
    row = lambda v: v.reshape(1, -1)
    ops = (states, act, We1a, We1b, row(be1), We2, row(be2), row(ge),
           row(bbe), We3, row(be3), Wn1x, Wn1a, Wn1g, row(bn1), Wn2,
           row(bn2), row(gn), row(bbn), Wn3, row(bn3), Jmat)
    in_specs = [pl.BlockSpec((BS, N, D), lambda b: (b, 0, 0)),
                pl.BlockSpec((1, 1, BS), lambda b: (b, 0, 0))]
    in_specs += [_full(o.shape) for o in ops[2:]]
    return pl.pallas_call(
        _block_kernel,
        grid=(G,),
        in_specs=in_specs,
        out_specs=pl.BlockSpec((BS, N, D), lambda b: (b, 0, 0)),
        out_shape=jax.ShapeDtypeStruct((B, N, D), jnp.float32),
        compiler_params=pltpu.CompilerParams(
            dimension_semantics=("parallel",)),
    )(*ops)


# speedup vs baseline: 1.0044x; 1.0044x over previous
import jax
import jax.numpy as jnp
from jax import lax
from jax.experimental import pallas as pl
from jax.experimental.pallas import tpu as pltpu

B, N, D, H, A = 1024, 16, 128, 128, 4
BS = 128
G = B // BS
EPS = 1e-5


def _ln(x, g, b):
    mu = jnp.mean(x, axis=-1, keepdims=True)
    var = jnp.mean((x - mu) ** 2, axis=-1, keepdims=True)
    return (x - mu) * lax.rsqrt(var + EPS) * g + b


def _edge_tail(E, J, ge, bbe):
    mu = jnp.dot(E, J, preferred_element_type=jnp.float32)
    d = E - mu
    var = jnp.mean(d * d, axis=-1, keepdims=True)
    return jax.nn.relu(d * (lax.rsqrt(var + EPS) * ge) + bbe)


def _block_kernel(x_ref, act_ref, we1a_ref, we1b_ref, be1_ref, we2_ref,
                  be2_ref, ge_ref, bbe_ref, we3_ref, be3_ref, wn1x_ref,
                  wn1a_ref, wn1g_ref, bn1_ref, wn2_ref, bn2_ref, gn_ref,
                  bbn_ref, wn3_ref, bn3_ref, j_ref, out_ref):
    f32 = jnp.float32
    Xf = x_ref[...].reshape(BS * N, D)
    J = j_ref[...]
    ge, bbe = ge_ref[...], bbe_ref[...]
    P = jnp.dot(Xf, we1a_ref[...], preferred_element_type=f32)
    Q = jnp.dot(Xf, we1b_ref[...], preferred_element_type=f32) + be1_ref[...]
    Fp = jnp.broadcast_to(P.reshape(BS, 1, N, H), (BS, N, N, H))
    Fq = jnp.broadcast_to(Q.reshape(BS, N, 1, H), (BS, N, N, H))
    Fd = (P + Q).reshape(BS, 1, N, H)
    F = jax.nn.relu(jnp.concatenate([Fp + Fq, Fd], axis=1))
    E = jnp.dot(F.reshape(BS * (N + 1) * N, H), we2_ref[...],
                preferred_element_type=f32) + be2_ref[...]
    T = _edge_tail(E, J, ge, bbe).reshape(BS, N + 1, N, H)
    S = (jnp.sum(T[:, :N], axis=1) - T[:, N]).reshape(BS * N, H)
    agg = jnp.dot(S, we3_ref[...], preferred_element_type=f32) \
        + (N - 1) * be3_ref[...]
    a = act_ref[...].reshape(BS, 1, 1)
    kk = lax.broadcasted_iota(jnp.int32, (BS, N, 8), 2)
    code = lax.broadcasted_iota(jnp.int32, (BS, N, 8), 1) * A + kk
    oh = jnp.where((code == a) & (kk < A), 1.0, 0.0).reshape(BS * N, 8)
    h = (jnp.dot(Xf, wn1x_ref[...], preferred_element_type=f32)
         + jnp.dot(oh, wn1a_ref[...], preferred_element_type=f32)
         + jnp.dot(agg, wn1g_ref[...], preferred_element_type=f32))
    h = jax.nn.relu(h + bn1_ref[...])
    h = jnp.dot(h, wn2_ref[...], preferred_element_type=f32) + bn2_ref[...]
    h = jax.nn.relu(_ln(h, gn_ref[...], bbn_ref[...]))
    out = jnp.dot(h, wn3_ref[...], preferred_element_type=f32) + bn3_ref[...]
    out_ref[...] = out.reshape(BS, N, D)


def _full(shape):
    nd = len(shape)
    return pl.BlockSpec(shape, lambda b: (0,) * nd)


def kernel(states, action, We1, be1, We2, be2, ge, bbe, We3, be3,
           Wn1, bn1, Wn2, bn2, gn, bbn, Wn3, bn3):
    act = action.reshape(G, 1, BS)
    We1a, We1b = We1[:D], We1[D:]
    Wn1x = Wn1[:D]
    Wn1a = jnp.zeros((8, H), jnp.float32).at[:A].set(Wn1[D:D + A])
    Wn1g = Wn1[D + A:]
    Jmat = jnp.full((H, H), 1.0 / H, jnp.float32)
    row = lambda v: v.reshape(1, -1)
    ops = (states, act, We1a, We1b, row(be1), We2, row(be2), row(ge),
           row(bbe), We3, row(be3), Wn1x, Wn1a, Wn1g, row(bn1), Wn2,
           row(bn2), row(gn), row(bbn), Wn3, row(bn3), Jmat)
    in_specs = [pl.BlockSpec((BS, N, D), lambda b: (b, 0, 0)),
                pl.BlockSpec((1, 1, BS), lambda b: (b, 0, 0))]
    in_specs += [_full(o.shape) for o in ops[2:]]
    return pl.pallas_call(
        _block_kernel,
        grid=(G,),
        in_specs=in_specs,
        out_specs=pl.BlockSpec((BS, N, D), lambda b: (b, 0, 0)),
        out_shape=jax.ShapeDtypeStruct((B, N, D), jnp.float32),
        compiler_params=pltpu.CompilerParams(
            dimension_semantics=("parallel",)),
    )(*ops)
